# baseline (speedup 1.0000x reference)
import functools

import jax
import jax.numpy as jnp
from jax import lax
from jax.experimental import pallas as pl
from jax.experimental.pallas import tpu as pltpu
from jax.experimental.pallas import tpu_sc as plsc

N = 10000
D_IN = 128
D_HID = 128
D_OUT = 64

NC = 2
NS = 16
NW = NC * NS

CHUNK = 128
IG = 8
N_PAD_ROWS = 112
N_ACC = N + N_PAD_ROWS
ZROWS = 632
ZCHUNK = 64


def _sc_propagate(hw, src2, dst2, zeros, d):
    cpw = src2.shape[1]
    nig = cpw // IG

    mesh = plsc.VectorSubcoreMesh(
        core_axis_name="c", subcore_axis_name="s", num_cores=NC, num_subcores=NS
    )

    @functools.partial(
        pl.kernel,
        out_type=jax.ShapeDtypeStruct((NC, N_ACC, d), jnp.float32),
        mesh=mesh,
        scratch_types=[
            pltpu.VMEM_SHARED((N_ACC, d), jnp.float32),
            pltpu.VMEM((2, IG, CHUNK), jnp.int32),
            pltpu.VMEM((2, IG, CHUNK), jnp.int32),
            pltpu.VMEM((2, CHUNK, d), jnp.float32),
            pltpu.VMEM((ZCHUNK, d), jnp.float32),
            [pltpu.SemaphoreType.DMA] * 2,
            [pltpu.SemaphoreType.DMA] * 2,
            [pltpu.SemaphoreType.DMA] * 2,
            pltpu.SemaphoreType.DMA,
        ],
    )
    def k(hw_hbm, src_hbm, dst_hbm, zeros_hbm, out_hbm, acc, src_v, dst_v, rbuf,
          zbuf, gsems, ssems, isems, zsem):
        c = lax.axis_index("c")
        s = lax.axis_index("s")
        wid = s * NC + c

        def idx_load(gi, slot, sem_id):
            sd = pltpu.make_async_copy(
                src_hbm.at[wid, pl.ds(gi * IG, IG)], src_v.at[slot], isems[sem_id])
            dd = pltpu.make_async_copy(
                dst_hbm.at[wid, pl.ds(gi * IG, IG)], dst_v.at[slot], isems[sem_id])
            return sd, dd

        def gather(slot, k_, b):
            return pltpu.make_async_copy(hw_hbm.at[src_v.at[slot, k_]],
                                         rbuf.at[b], gsems[b])

        def scatter(slot, k_, b):
            return pltpu.make_async_copy(rbuf.at[b], acc.at[dst_v.at[slot, k_]],
                                         ssems[b])

        zstage = pltpu.make_async_copy(zeros_hbm, zbuf, zsem)
        zstage.start()
        for dsc in idx_load(0, 0, 0):
            dsc.start()
        for dsc in idx_load(0, 0, 0):
            dsc.wait()
        gather(0, 0, 0).start()
        gather(0, 1, 1).start()
        zstage.wait()
        nfull = ZROWS // ZCHUNK
        ztail = ZROWS - nfull * ZCHUNK
        zouts = [
            pltpu.make_async_copy(
                zbuf, acc.at[pl.ds(s * ZROWS + i * ZCHUNK, ZCHUNK)], zsem)
            for i in range(nfull)
        ] + [
            pltpu.make_async_copy(
                zbuf.at[pl.ds(0, ztail)],
                acc.at[pl.ds(s * ZROWS + nfull * ZCHUNK, ztail)], zsem)
        ]
        for dsc in zouts:
            dsc.start()
        for dsc in zouts:
            dsc.wait()
        plsc.subcore_barrier()

        def group(slot, gi, first, last):
            nslot = 1 - slot
            for k_ in range(IG):
                b = k_ % 2
                if k_ == 0 and not last:
                    for dsc in idx_load(gi + 1, nslot, nslot):
                        dsc.start()
                if k_ == IG - 3 and not last:
                    for dsc in idx_load(gi + 1, nslot, nslot):
                        dsc.wait()
                gather(slot, k_, b).wait()
                scatter(slot, k_, b).start(add=True)
                scatter(slot, k_, b).wait()
                if k_ >= IG - 2:
                    if not last:
                        gather(nslot, k_ - (IG - 2), b).start()
                else:
                    gather(slot, k_ + 2, b).start()

        def pair(t, carry):
            gi = 2 * t
            group(0, gi, False, False)
            group(1, gi + 1, False, False)
            return carry

        group(0, 0, True, False)
        group(1, 1, False, False)
        lax.fori_loop(1, nig // 2 - 1, pair, 0)
        group(0, nig - 2, False, False)
        group(1, nig - 1, False, True)
        plsc.subcore_barrier()

        pltpu.sync_copy(
            acc.at[pl.ds(s * ZROWS, ZROWS)],
            out_hbm.at[c, pl.ds(s * ZROWS, ZROWS)],
        )

    return k(hw, src2, dst2, zeros)


def _mm_first(x, w):
    bm = 1000
    d_in, d_out = w.shape

    def body(x_ref, w_ref, o_ref):
        o_ref[...] = jnp.dot(x_ref[...], w_ref[...], preferred_element_type=jnp.float32)

    return pl.pallas_call(
        body,
        grid=(N // bm,),
        in_specs=[
            pl.BlockSpec((bm, d_in), lambda i: (i, 0)),
            pl.BlockSpec((d_in, d_out), lambda i: (0, 0)),
        ],
        out_specs=pl.BlockSpec((bm, d_out), lambda i: (i, 0)),
        out_shape=jax.ShapeDtypeStruct((N, d_out), jnp.float32),
    )(x, w)


def _mm_fused(p, b, w):
    bm = 1000
    d_in, d_out = w.shape
    b2 = b.reshape(1, d_in)

    def body(p_ref, b_ref, w_ref, o_ref):
        h = jnp.maximum(p_ref[0] + p_ref[1] + b_ref[...], 0.0)
        o_ref[...] = jnp.dot(h, w_ref[...], preferred_element_type=jnp.float32)

    return pl.pallas_call(
        body,
        grid=(N // bm,),
        in_specs=[
            pl.BlockSpec((2, bm, d_in), lambda i: (0, i, 0)),
            pl.BlockSpec((1, d_in), lambda i: (0, 0)),
            pl.BlockSpec((d_in, d_out), lambda i: (0, 0)),
        ],
        out_specs=pl.BlockSpec((bm, d_out), lambda i: (i, 0)),
        out_shape=jax.ShapeDtypeStruct((N, d_out), jnp.float32),
    )(p, b2, w)


def _final_combine(p, b):
    bm = 1000
    d = p.shape[-1]
    b2 = b.reshape(1, D_OUT)

    def body(p_ref, b_ref, o_ref):
        v = p_ref[0] + p_ref[1]
        o_ref[...] = v[:, :D_OUT] + b_ref[...]

    return pl.pallas_call(
        body,
        grid=(N // bm,),
        in_specs=[
            pl.BlockSpec((2, bm, d), lambda i: (0, i, 0)),
            pl.BlockSpec((1, D_OUT), lambda i: (0, 0)),
        ],
        out_specs=pl.BlockSpec((bm, D_OUT), lambda i: (i, 0)),
        out_shape=jax.ShapeDtypeStruct((N, D_OUT), jnp.float32),
    )(p, b2)


def kernel(x, edge_index, W1, b1, W2, b2, W3, b3):
    e = edge_index.shape[1]
    per_group = CHUNK * NW * 2 * IG
    e_pad = ((e + per_group - 1) // per_group) * per_group
    npad = e_pad - e

    src = edge_index[0]
    dst = edge_index[1]
    pad_ids = jnp.arange(npad, dtype=jnp.int32)
    src2 = jnp.concatenate([src, pad_ids % N]).reshape(NW, -1, CHUNK)
    dst2 = jnp.concatenate([dst, N + pad_ids % N_PAD_ROWS]).reshape(NW, -1, CHUNK)

    zeros128 = jnp.zeros((ZCHUNK, D_HID), jnp.float32)
    w3p = jnp.pad(W3, ((0, 0), (0, D_HID - D_OUT)))

    h = _mm_first(x, W1)
    p = _sc_propagate(h, src2, dst2, zeros128, D_HID)
    h = _mm_fused(p, b1, W2)
    p = _sc_propagate(h, src2, dst2, zeros128, D_HID)
    h = _mm_fused(p, b2, w3p)
    p = _sc_propagate(h, src2, dst2, zeros128, D_HID)
    return _final_combine(p, b3)


# speedup vs baseline: 1.0276x; 1.0276x over previous
import functools

import jax
import jax.numpy as jnp
from jax import lax
from jax.experimental import pallas as pl
from jax.experimental.pallas import tpu as pltpu
from jax.experimental.pallas import tpu_sc as plsc

N = 10000
D_IN = 128
D_HID = 128
D_OUT = 64

NC = 2
NS = 16
NW = NC * NS

CHUNK = 128
IG = 8
N_PAD_ROWS = 112
N_ACC = N + N_PAD_ROWS
ZROWS = 632
ZCHUNK = 64


def _sc_propagate(hw, src2, dst2, zeros, d):
    cpw = src2.shape[1]
    nig = cpw // IG

    mesh = plsc.VectorSubcoreMesh(
        core_axis_name="c", subcore_axis_name="s", num_cores=NC, num_subcores=NS
    )

    @functools.partial(
        pl.kernel,
        out_type=jax.ShapeDtypeStruct((NC, N_ACC, d), jnp.float32),
        mesh=mesh,
        scratch_types=[
            pltpu.VMEM_SHARED((N_ACC, d), jnp.float32),
            pltpu.VMEM((2, IG, CHUNK), jnp.int32),
            pltpu.VMEM((2, IG, CHUNK), jnp.int32),
            pltpu.VMEM((2, CHUNK, d), jnp.float32),
            pltpu.VMEM((ZCHUNK, d), jnp.float32),
            [pltpu.SemaphoreType.DMA] * 2,
            [pltpu.SemaphoreType.DMA] * 2,
            [pltpu.SemaphoreType.DMA] * 2,
            pltpu.SemaphoreType.DMA,
        ],
    )
    def k(hw_hbm, src_hbm, dst_hbm, zeros_hbm, out_hbm, acc, src_v, dst_v, rbuf,
          zbuf, gsems, ssems, isems, zsem):
        c = lax.axis_index("c")
        s = lax.axis_index("s")
        wid = s * NC + c

        def idx_load(gi, slot, sem_id):
            sd = pltpu.make_async_copy(
                src_hbm.at[wid, pl.ds(gi * IG, IG)], src_v.at[slot], isems[sem_id])
            dd = pltpu.make_async_copy(
                dst_hbm.at[wid, pl.ds(gi * IG, IG)], dst_v.at[slot], isems[sem_id])
            return sd, dd

        def gather(slot, k_, b):
            return pltpu.make_async_copy(hw_hbm.at[src_v.at[slot, k_]],
                                         rbuf.at[b], gsems[b])

        def scatter(slot, k_, b):
            return pltpu.make_async_copy(rbuf.at[b], acc.at[dst_v.at[slot, k_]],
                                         ssems[b])

        zstage = pltpu.make_async_copy(zeros_hbm, zbuf, zsem)
        zstage.start()
        for dsc in idx_load(0, 0, 0):
            dsc.start()
        for dsc in idx_load(0, 0, 0):
            dsc.wait()
        gather(0, 0, 0).start()
        gather(0, 1, 1).start()
        zstage.wait()
        nfull = ZROWS // ZCHUNK
        ztail = ZROWS - nfull * ZCHUNK
        zouts = [
            pltpu.make_async_copy(
                zbuf, acc.at[pl.ds(s * ZROWS + i * ZCHUNK, ZCHUNK)], zsem)
            for i in range(nfull)
        ] + [
            pltpu.make_async_copy(
                zbuf.at[pl.ds(0, ztail)],
                acc.at[pl.ds(s * ZROWS + nfull * ZCHUNK, ztail)], zsem)
        ]
        for dsc in zouts:
            dsc.start()
        for dsc in zouts:
            dsc.wait()
        plsc.subcore_barrier()

        def group(slot, gi, first, last):
            nslot = 1 - slot
            for k_ in range(IG):
                b = k_ % 2
                if k_ == 0 and not last:
                    for dsc in idx_load(gi + 1, nslot, nslot):
                        dsc.start()
                if k_ == IG - 3 and not last:
                    for dsc in idx_load(gi + 1, nslot, nslot):
                        dsc.wait()
                gather(slot, k_, b).wait()
                scatter(slot, k_, b).start(add=True)
                scatter(slot, k_, b).wait()
                if k_ >= IG - 2:
                    if not last:
                        gather(nslot, k_ - (IG - 2), b).start()
                else:
                    gather(slot, k_ + 2, b).start()

        def pair(t, carry):
            gi = 2 * t
            group(0, gi, False, False)
            group(1, gi + 1, False, False)
            return carry

        group(0, 0, True, False)
        group(1, 1, False, False)
        lax.fori_loop(1, nig // 2 - 1, pair, 0)
        group(0, nig - 2, False, False)
        group(1, nig - 1, False, True)
        plsc.subcore_barrier()

        pltpu.sync_copy(
            acc.at[pl.ds(s * ZROWS, ZROWS)],
            out_hbm.at[c, pl.ds(s * ZROWS, ZROWS)],
        )

    return k(hw, src2, dst2, zeros)


def _mm_first(x, w):
    bm = 2000
    d_in, d_out = w.shape

    def body(x_ref, w_ref, o_ref):
        o_ref[...] = jnp.dot(x_ref[...], w_ref[...], preferred_element_type=jnp.float32)

    return pl.pallas_call(
        body,
        grid=(N // bm,),
        in_specs=[
            pl.BlockSpec((bm, d_in), lambda i: (i, 0)),
            pl.BlockSpec((d_in, d_out), lambda i: (0, 0)),
        ],
        out_specs=pl.BlockSpec((bm, d_out), lambda i: (i, 0)),
        out_shape=jax.ShapeDtypeStruct((N, d_out), jnp.float32),
    )(x, w)


def _mm_fused(p, b, w):
    bm = 2000
    d_in, d_out = w.shape
    b2 = b.reshape(1, d_in)

    def body(p_ref, b_ref, w_ref, o_ref):
        h = jnp.maximum(p_ref[0] + p_ref[1] + b_ref[...], 0.0)
        o_ref[...] = jnp.dot(h, w_ref[...], preferred_element_type=jnp.float32)

    return pl.pallas_call(
        body,
        grid=(N // bm,),
        in_specs=[
            pl.BlockSpec((2, bm, d_in), lambda i: (0, i, 0)),
            pl.BlockSpec((1, d_in), lambda i: (0, 0)),
            pl.BlockSpec((d_in, d_out), lambda i: (0, 0)),
        ],
        out_specs=pl.BlockSpec((bm, d_out), lambda i: (i, 0)),
        out_shape=jax.ShapeDtypeStruct((N, d_out), jnp.float32),
    )(p, b2, w)


def _final_combine(p, b):
    bm = 2000
    d = p.shape[-1]
    b2 = b.reshape(1, D_OUT)

    def body(p_ref, b_ref, o_ref):
        v = p_ref[0] + p_ref[1]
        o_ref[...] = v[:, :D_OUT] + b_ref[...]

    return pl.pallas_call(
        body,
        grid=(N // bm,),
        in_specs=[
            pl.BlockSpec((2, bm, d), lambda i: (0, i, 0)),
            pl.BlockSpec((1, D_OUT), lambda i: (0, 0)),
        ],
        out_specs=pl.BlockSpec((bm, D_OUT), lambda i: (i, 0)),
        out_shape=jax.ShapeDtypeStruct((N, D_OUT), jnp.float32),
    )(p, b2)


def kernel(x, edge_index, W1, b1, W2, b2, W3, b3):
    e = edge_index.shape[1]
    per_group = CHUNK * NW * 2 * IG
    e_pad = ((e + per_group - 1) // per_group) * per_group
    npad = e_pad - e

    src = edge_index[0]
    dst = edge_index[1]
    pad_ids = jnp.arange(npad, dtype=jnp.int32)
    src2 = jnp.concatenate([src, pad_ids % N]).reshape(NW, -1, CHUNK)
    dst2 = jnp.concatenate([dst, N + pad_ids % N_PAD_ROWS]).reshape(NW, -1, CHUNK)

    zeros128 = jnp.zeros((ZCHUNK, D_HID), jnp.float32)
    w3p = jnp.pad(W3, ((0, 0), (0, D_HID - D_OUT)))

    h = _mm_first(x, W1)
    p = _sc_propagate(h, src2, dst2, zeros128, D_HID)
    h = _mm_fused(p, b1, W2)
    p = _sc_propagate(h, src2, dst2, zeros128, D_HID)
    h = _mm_fused(p, b2, w3p)
    p = _sc_propagate(h, src2, dst2, zeros128, D_HID)
    return _final_combine(p, b3)


# speedup vs baseline: 1.0503x; 1.0221x over previous
import functools

import jax
import jax.numpy as jnp
from jax import lax
from jax.experimental import pallas as pl
from jax.experimental.pallas import tpu as pltpu
from jax.experimental.pallas import tpu_sc as plsc

N = 10000
D_IN = 128
D_HID = 128
D_OUT = 64

NC = 2
NS = 16
NW = NC * NS

CHUNK = 128
IG = 8
N_PAD_ROWS = 112
N_ACC = N + N_PAD_ROWS
ZROWS = 632
ZCHUNK = 64


def _sc_propagate(hw, src2, dst2, zeros, d):
    cpw = src2.shape[1]
    nig = cpw // IG

    mesh = plsc.VectorSubcoreMesh(
        core_axis_name="c", subcore_axis_name="s", num_cores=NC, num_subcores=NS
    )

    @functools.partial(
        pl.kernel,
        out_type=jax.ShapeDtypeStruct((NC, N_ACC, d), jnp.float32),
        mesh=mesh,
        scratch_types=[
            pltpu.VMEM_SHARED((N_ACC, d), jnp.float32),
            pltpu.VMEM((2, IG, CHUNK), jnp.int32),
            pltpu.VMEM((2, IG, CHUNK), jnp.int32),
            pltpu.VMEM((2, CHUNK, d), jnp.float32),
            pltpu.VMEM((ZCHUNK, d), jnp.float32),
            [pltpu.SemaphoreType.DMA] * 2,
            [pltpu.SemaphoreType.DMA] * 2,
            [pltpu.SemaphoreType.DMA] * 2,
            pltpu.SemaphoreType.DMA,
        ],
    )
    def k(hw_hbm, src_hbm, dst_hbm, zeros_hbm, out_hbm, acc, src_v, dst_v, rbuf,
          zbuf, gsems, ssems, isems, zsem):
        c = lax.axis_index("c")
        s = lax.axis_index("s")
        wid = s * NC + c

        def idx_load(gi, slot, sem_id):
            sd = pltpu.make_async_copy(
                src_hbm.at[wid, pl.ds(gi * IG, IG)], src_v.at[slot], isems[sem_id])
            dd = pltpu.make_async_copy(
                dst_hbm.at[wid, pl.ds(gi * IG, IG)], dst_v.at[slot], isems[sem_id])
            return sd, dd

        def gather(slot, k_, b):
            return pltpu.make_async_copy(hw_hbm.at[src_v.at[slot, k_]],
                                         rbuf.at[b], gsems[b])

        def scatter(slot, k_, b):
            return pltpu.make_async_copy(rbuf.at[b], acc.at[dst_v.at[slot, k_]],
                                         ssems[b])

        zstage = pltpu.make_async_copy(zeros_hbm, zbuf, zsem)
        zstage.start()
        for dsc in idx_load(0, 0, 0):
            dsc.start()
        for dsc in idx_load(0, 0, 0):
            dsc.wait()
        gather(0, 0, 0).start()
        gather(0, 1, 1).start()
        zstage.wait()
        nfull = ZROWS // ZCHUNK
        ztail = ZROWS - nfull * ZCHUNK
        zouts = [
            pltpu.make_async_copy(
                zbuf, acc.at[pl.ds(s * ZROWS + i * ZCHUNK, ZCHUNK)], zsem)
            for i in range(nfull)
        ] + [
            pltpu.make_async_copy(
                zbuf.at[pl.ds(0, ztail)],
                acc.at[pl.ds(s * ZROWS + nfull * ZCHUNK, ztail)], zsem)
        ]
        for dsc in zouts:
            dsc.start()
        for dsc in zouts:
            dsc.wait()
        plsc.subcore_barrier()

        def group(slot, gi, first, last):
            nslot = 1 - slot
            for k_ in range(IG):
                b = k_ % 2
                if k_ == 0 and not last:
                    for dsc in idx_load(gi + 1, nslot, nslot):
                        dsc.start()
                if k_ == IG - 3 and not last:
                    for dsc in idx_load(gi + 1, nslot, nslot):
                        dsc.wait()
                gather(slot, k_, b).wait()
                scatter(slot, k_, b).start(add=True)
                scatter(slot, k_, b).wait()
                if k_ >= IG - 2:
                    if not last:
                        gather(nslot, k_ - (IG - 2), b).start()
                else:
                    gather(slot, k_ + 2, b).start()

        def pair(t, carry):
            gi = 2 * t
            group(0, gi, False, False)
            group(1, gi + 1, False, False)
            return carry

        group(0, 0, True, False)
        group(1, 1, False, False)
        lax.fori_loop(1, nig // 2 - 1, pair, 0)
        group(0, nig - 2, False, False)
        group(1, nig - 1, False, True)
        plsc.subcore_barrier()

        pltpu.sync_copy(
            acc.at[pl.ds(s * ZROWS, ZROWS)],
            out_hbm.at[c, pl.ds(s * ZROWS, ZROWS)],
        )

    return k(hw, src2, dst2, zeros)


def _mm_first(x, w):
    bm = 5000
    d_in, d_out = w.shape

    def body(x_ref, w_ref, o_ref):
        o_ref[...] = jnp.dot(x_ref[...], w_ref[...], preferred_element_type=jnp.float32)

    return pl.pallas_call(
        body,
        grid=(N // bm,),
        in_specs=[
            pl.BlockSpec((bm, d_in), lambda i: (i, 0)),
            pl.BlockSpec((d_in, d_out), lambda i: (0, 0)),
        ],
        out_specs=pl.BlockSpec((bm, d_out), lambda i: (i, 0)),
        out_shape=jax.ShapeDtypeStruct((N, d_out), jnp.float32),
    )(x, w)


def _mm_fused(p, b, w):
    bm = 5000
    d_in, d_out = w.shape
    b2 = b.reshape(1, d_in)

    def body(p_ref, b_ref, w_ref, o_ref):
        h = jnp.maximum(p_ref[0] + p_ref[1] + b_ref[...], 0.0)
        o_ref[...] = jnp.dot(h, w_ref[...], preferred_element_type=jnp.float32)

    return pl.pallas_call(
        body,
        grid=(N // bm,),
        in_specs=[
            pl.BlockSpec((2, bm, d_in), lambda i: (0, i, 0)),
            pl.BlockSpec((1, d_in), lambda i: (0, 0)),
            pl.BlockSpec((d_in, d_out), lambda i: (0, 0)),
        ],
        out_specs=pl.BlockSpec((bm, d_out), lambda i: (i, 0)),
        out_shape=jax.ShapeDtypeStruct((N, d_out), jnp.float32),
    )(p, b2, w)


def _final_combine(p, b):
    bm = 5000
    d = p.shape[-1]
    b2 = b.reshape(1, D_OUT)

    def body(p_ref, b_ref, o_ref):
        v = p_ref[0] + p_ref[1]
        o_ref[...] = v[:, :D_OUT] + b_ref[...]

    return pl.pallas_call(
        body,
        grid=(N // bm,),
        in_specs=[
            pl.BlockSpec((2, bm, d), lambda i: (0, i, 0)),
            pl.BlockSpec((1, D_OUT), lambda i: (0, 0)),
        ],
        out_specs=pl.BlockSpec((bm, D_OUT), lambda i: (i, 0)),
        out_shape=jax.ShapeDtypeStruct((N, D_OUT), jnp.float32),
    )(p, b2)


def kernel(x, edge_index, W1, b1, W2, b2, W3, b3):
    e = edge_index.shape[1]
    per_group = CHUNK * NW * 2 * IG
    e_pad = ((e + per_group - 1) // per_group) * per_group
    npad = e_pad - e

    src = edge_index[0]
    dst = edge_index[1]
    pad_ids = jnp.arange(npad, dtype=jnp.int32)
    src2 = jnp.concatenate([src, pad_ids % N]).reshape(NW, -1, CHUNK)
    dst2 = jnp.concatenate([dst, N + pad_ids % N_PAD_ROWS]).reshape(NW, -1, CHUNK)

    zeros128 = jnp.zeros((ZCHUNK, D_HID), jnp.float32)
    w3p = jnp.pad(W3, ((0, 0), (0, D_HID - D_OUT)))

    h = _mm_first(x, W1)
    p = _sc_propagate(h, src2, dst2, zeros128, D_HID)
    h = _mm_fused(p, b1, W2)
    p = _sc_propagate(h, src2, dst2, zeros128, D_HID)
    h = _mm_fused(p, b2, w3p)
    p = _sc_propagate(h, src2, dst2, zeros128, D_HID)
    return _final_combine(p, b3)
